# baseline (speedup 1.0000x reference)
import jax
import jax.numpy as jnp
import numpy as np
from jax.experimental import pallas as pl
from jax.experimental.pallas import tpu as pltpu

_INV_EM1 = float(1.0 / (np.exp(1.0) - 1.0))
_L = 128
_K = 256


def _vel_kernel(pos_ref, vel_ref, velout_ref, gamma_ref):
    x = pos_ref[:, 0, :]
    y = pos_ref[:, 1, :]
    vx = vel_ref[:, 0, :]
    vy = vel_ref[:, 1, :]
    m = (x >= 0.0) & (x <= 0.25) & (y >= 0.0) & (y <= 1.0)
    xr = jnp.clip(x * 4.0, 0.0, 1.0)
    t = jnp.exp(jnp.log(xr) * 3.5)
    g = (jnp.exp(t) - 1.0) * _INV_EM1
    g = jnp.minimum(g, 1.0)
    velout_ref[:, 0, :] = jnp.where(m, vx + g * (1.0 - vx), vx)
    velout_ref[:, 1, :] = jnp.where(m, vy * (1.0 - g), vy)
    gamma_ref[...] = g.reshape(_K * _L)


def kernel(fluidPosition, fluidVelocity, fluidArea):
    n = fluidPosition.shape[0]
    nk = n // _L
    pos = fluidPosition.reshape(nk, _L, 2).swapaxes(1, 2)
    vel = fluidVelocity.reshape(nk, _L, 2).swapaxes(1, 2)
    grid = (nk + _K - 1) // _K
    vel_out, gamma = pl.pallas_call(
        _vel_kernel,
        grid=(grid,),
        in_specs=[
            pl.BlockSpec((_K, 2, _L), lambda i: (i, 0, 0)),
            pl.BlockSpec((_K, 2, _L), lambda i: (i, 0, 0)),
        ],
        out_specs=[
            pl.BlockSpec((_K, 2, _L), lambda i: (i, 0, 0)),
            pl.BlockSpec((_K * _L,), lambda i: (i,)),
        ],
        out_shape=[
            jax.ShapeDtypeStruct((nk, 2, _L), jnp.float32),
            jax.ShapeDtypeStruct((n,), jnp.float32),
        ],
    )(pos, vel)
    vel_out = vel_out.swapaxes(1, 2).reshape(n, 2)
    return vel_out, gamma


# speedup vs baseline: 1.0431x; 1.0431x over previous
import jax
import jax.numpy as jnp
import numpy as np
from jax.experimental import pallas as pl

_INV_EM1 = float(1.0 / (np.exp(1.0) - 1.0))
_L = 128
_K = 512


def _vel_kernel(posx_ref, posy_ref, velx_ref, vely_ref,
                velout_ref, gamma_ref):
    x = posx_ref[...].reshape(_K, _L)
    y = posy_ref[...].reshape(_K, _L)
    vx = velx_ref[...].reshape(_K, _L)
    vy = vely_ref[...].reshape(_K, _L)
    m = (x >= 0.0) & (x <= 0.25) & (y >= 0.0) & (y <= 1.0)
    xr = jnp.clip(x * 4.0, 0.0, 1.0)
    t = jnp.exp(jnp.log(xr) * 3.5)
    g = (jnp.exp(t) - 1.0) * _INV_EM1
    g = jnp.minimum(g, 1.0)
    vx_out = jnp.where(m, vx + g * (1.0 - vx), vx)
    vy_out = jnp.where(m, vy * (1.0 - g), vy)
    v_out = jnp.stack([vx_out, vy_out], axis=1)
    velout_ref[...] = v_out.reshape(_K, 2, 1, _L)
    gamma_ref[...] = g.reshape(_K * _L)


def kernel(fluidPosition, fluidVelocity, fluidArea):
    n = fluidPosition.shape[0]
    nk = n // _L
    pos = fluidPosition.reshape(nk, _L, 2).swapaxes(1, 2).reshape(nk, 2, 1, _L)
    vel = fluidVelocity.reshape(nk, _L, 2).swapaxes(1, 2).reshape(nk, 2, 1, _L)
    grid = (nk + _K - 1) // _K
    plane = lambda c: pl.BlockSpec((_K, 1, 1, _L), lambda i, c=c: (i, c, 0, 0))
    vel_out, gamma = pl.pallas_call(
        _vel_kernel,
        grid=(grid,),
        in_specs=[plane(0), plane(1), plane(0), plane(1)],
        out_specs=[
            pl.BlockSpec((_K, 2, 1, _L), lambda i: (i, 0, 0, 0)),
            pl.BlockSpec((_K * _L,), lambda i: (i,)),
        ],
        out_shape=[
            jax.ShapeDtypeStruct((nk, 2, 1, _L), jnp.float32),
            jax.ShapeDtypeStruct((n,), jnp.float32),
        ],
    )(pos, pos, vel, vel)
    vel_out = vel_out.reshape(nk, 2, _L).swapaxes(1, 2).reshape(n, 2)
    return vel_out, gamma


# speedup vs baseline: 1.5087x; 1.4463x over previous
import jax
import jax.numpy as jnp
import numpy as np
from jax.experimental import pallas as pl
from jax.experimental.pallas import tpu as pltpu

_INV_EM1 = float(1.0 / (np.exp(1.0) - 1.0))
_L = 128
_K = 512


def _vel_kernel(pos_ref, vel_ref, velout_ref, gamma_ref,
                sx, sy, svx, svy, sem_in, sem_out):
    cps = [
        pltpu.make_async_copy(pos_ref.at[:, 0, :], sx, sem_in),
        pltpu.make_async_copy(pos_ref.at[:, 1, :], sy, sem_in),
        pltpu.make_async_copy(vel_ref.at[:, 0, :], svx, sem_in),
        pltpu.make_async_copy(vel_ref.at[:, 1, :], svy, sem_in),
    ]
    for cp in cps:
        cp.start()
    for cp in cps:
        cp.wait()
    x = sx[...]
    y = sy[...]
    vx = svx[...]
    vy = svy[...]
    m = (x >= 0.0) & (x <= 0.25) & (y >= 0.0) & (y <= 1.0)
    xr = jnp.clip(x * 4.0, 0.0, 1.0)
    t = jnp.exp(jnp.log(xr) * 3.5)
    g = (jnp.exp(t) - 1.0) * _INV_EM1
    g = jnp.minimum(g, 1.0)
    svx[...] = jnp.where(m, vx + g * (1.0 - vx), vx)
    svy[...] = jnp.where(m, vy * (1.0 - g), vy)
    gamma_ref[...] = g.reshape(_K * _L)
    ocs = [
        pltpu.make_async_copy(svx, velout_ref.at[:, 0, :], sem_out),
        pltpu.make_async_copy(svy, velout_ref.at[:, 1, :], sem_out),
    ]
    for cp in ocs:
        cp.start()
    for cp in ocs:
        cp.wait()


def kernel(fluidPosition, fluidVelocity, fluidArea):
    n = fluidPosition.shape[0]
    nk = n // _L
    pos = fluidPosition.reshape(nk, _L, 2).swapaxes(1, 2)
    vel = fluidVelocity.reshape(nk, _L, 2).swapaxes(1, 2)
    grid = (nk + _K - 1) // _K
    vel_out, gamma = pl.pallas_call(
        _vel_kernel,
        grid=(grid,),
        in_specs=[
            pl.BlockSpec((_K, 2, _L), lambda i: (i, 0, 0)),
            pl.BlockSpec((_K, 2, _L), lambda i: (i, 0, 0)),
        ],
        out_specs=[
            pl.BlockSpec((_K, 2, _L), lambda i: (i, 0, 0)),
            pl.BlockSpec((_K * _L,), lambda i: (i,)),
        ],
        out_shape=[
            jax.ShapeDtypeStruct((nk, 2, _L), jnp.float32),
            jax.ShapeDtypeStruct((n,), jnp.float32),
        ],
        scratch_shapes=[
            pltpu.VMEM((_K, _L), jnp.float32),
            pltpu.VMEM((_K, _L), jnp.float32),
            pltpu.VMEM((_K, _L), jnp.float32),
            pltpu.VMEM((_K, _L), jnp.float32),
            pltpu.SemaphoreType.DMA,
            pltpu.SemaphoreType.DMA,
        ],
    )(pos, vel)
    vel_out = vel_out.swapaxes(1, 2).reshape(n, 2)
    return vel_out, gamma


# speedup vs baseline: 1.6119x; 1.0684x over previous
import jax
import jax.numpy as jnp
import numpy as np
from jax.experimental import pallas as pl
from jax.experimental.pallas import tpu as pltpu

_INV_EM1 = float(1.0 / (np.exp(1.0) - 1.0))
_L = 128
_K = 1024


def _vel_kernel(pos_ref, vel_ref, velout_ref, gamma_ref,
                sx, sy, svx, svy, sem_in, sem_out):
    cps = [
        pltpu.make_async_copy(pos_ref.at[:, 0, :], sx, sem_in),
        pltpu.make_async_copy(pos_ref.at[:, 1, :], sy, sem_in),
        pltpu.make_async_copy(vel_ref.at[:, 0, :], svx, sem_in),
        pltpu.make_async_copy(vel_ref.at[:, 1, :], svy, sem_in),
    ]
    for cp in cps:
        cp.start()
    for cp in cps:
        cp.wait()
    x = sx[...]
    y = sy[...]
    vx = svx[...]
    vy = svy[...]
    m = (x >= 0.0) & (x <= 0.25) & (y >= 0.0) & (y <= 1.0)
    xr = jnp.clip(x * 4.0, 0.0, 1.0)
    t = jnp.exp(jnp.log(xr) * 3.5)
    g = (jnp.exp(t) - 1.0) * _INV_EM1
    g = jnp.minimum(g, 1.0)
    svx[...] = jnp.where(m, vx + g * (1.0 - vx), vx)
    svy[...] = jnp.where(m, vy * (1.0 - g), vy)
    gamma_ref[...] = g.reshape(_K * _L)
    ocs = [
        pltpu.make_async_copy(svx, velout_ref.at[:, 0, :], sem_out),
        pltpu.make_async_copy(svy, velout_ref.at[:, 1, :], sem_out),
    ]
    for cp in ocs:
        cp.start()
    for cp in ocs:
        cp.wait()


def kernel(fluidPosition, fluidVelocity, fluidArea):
    n = fluidPosition.shape[0]
    nk = n // _L
    pos = fluidPosition.reshape(nk, _L, 2).swapaxes(1, 2)
    vel = fluidVelocity.reshape(nk, _L, 2).swapaxes(1, 2)
    grid = (nk + _K - 1) // _K
    vel_out, gamma = pl.pallas_call(
        _vel_kernel,
        grid=(grid,),
        in_specs=[
            pl.BlockSpec((_K, 2, _L), lambda i: (i, 0, 0)),
            pl.BlockSpec((_K, 2, _L), lambda i: (i, 0, 0)),
        ],
        out_specs=[
            pl.BlockSpec((_K, 2, _L), lambda i: (i, 0, 0)),
            pl.BlockSpec((_K * _L,), lambda i: (i,)),
        ],
        out_shape=[
            jax.ShapeDtypeStruct((nk, 2, _L), jnp.float32),
            jax.ShapeDtypeStruct((n,), jnp.float32),
        ],
        scratch_shapes=[
            pltpu.VMEM((_K, _L), jnp.float32),
            pltpu.VMEM((_K, _L), jnp.float32),
            pltpu.VMEM((_K, _L), jnp.float32),
            pltpu.VMEM((_K, _L), jnp.float32),
            pltpu.SemaphoreType.DMA,
            pltpu.SemaphoreType.DMA,
        ],
    )(pos, vel)
    vel_out = vel_out.swapaxes(1, 2).reshape(n, 2)
    return vel_out, gamma
